# Initial kernel scaffold; baseline (speedup 1.0000x reference)
import jax
import jax.numpy as jnp
from jax.experimental import pallas as pl


def kernel(x, enc_W0, enc_b0, enc_W1, enc_b1, enc_W2, enc_b2, venc_Wmu, venc_bmu, venc_Wlv, venc_blv, dec_W0, dec_b0, dec_W1, dec_b1, dec_W2, dec_b2, vdec_Wmu, vdec_bmu, vdec_Wsig, vdec_bsig):
    raise NotImplementedError("write your pallas kernel here")


# speedup vs baseline: 2.8693x; 2.8693x over previous
import jax
import jax.numpy as jnp
from jax.experimental import pallas as pl
from jax.experimental.pallas import tpu as pltpu

_N = 64
_B = 512
_F = 512
_Z = 256
_BB = 8


def _roll_sum(h):
    left = jnp.concatenate([h[1:], h[:1]], axis=0)
    right = jnp.concatenate([h[-1:], h[:-1]], axis=0)
    return left + right


def _net_kernel(x_ref, eps_ref,
                e0s, e0n, e0b, e1s, e1n, e1b, e2s, e2n, e2b,
                wmu, bmu, wlv, blv,
                d0s, d0n, d0b, d1s, d1n, d1b, d2s, d2n, d2b,
                wdm, bdm, wds, bds,
                out_ref):
    n, b, f = x_ref.shape
    m = n * b

    def gnn(h, ws, wn, bias):
        k = h.shape[-1]
        s = _roll_sum(h) * 0.5
        y = jnp.dot(h.reshape(m, k), ws[...], preferred_element_type=jnp.float32)
        y = y + jnp.dot(s.reshape(m, k), wn[...], preferred_element_type=jnp.float32)
        y = jnp.maximum(y + bias[...], 0.0)
        return y.reshape(n, b, y.shape[-1])

    h = x_ref[...]
    h = gnn(h, e0s, e0n, e0b)
    h = gnn(h, e1s, e1n, e1b)
    h = gnn(h, e2s, e2n, e2b)

    h2 = h.reshape(m, h.shape[-1])
    mu = jnp.dot(h2, wmu[...], preferred_element_type=jnp.float32) + bmu[...]
    lv = jnp.dot(h2, wlv[...], preferred_element_type=jnp.float32) + blv[...]
    z = mu + jnp.exp(0.5 * lv) * eps_ref[...].reshape(m, _Z)
    h = z.reshape(n, b, _Z)

    h = gnn(h, d0s, d0n, d0b)
    h = gnn(h, d1s, d1n, d1b)
    h = gnn(h, d2s, d2n, d2b)

    h2 = h.reshape(m, h.shape[-1])
    dmu = jnp.dot(h2, wdm[...], preferred_element_type=jnp.float32) + bdm[...]
    dsg = jax.nn.softplus(
        jnp.dot(h2, wds[...], preferred_element_type=jnp.float32) + bds[...])
    out_ref[...] = jnp.exp(dmu + 0.5 * dsg * dsg).reshape(n, b, f)


def _pad2(w, rows, cols):
    r, c = w.shape
    return jnp.pad(w, ((0, rows - r), (0, cols - c)))


def _pad_b(v, cols):
    return jnp.pad(v, (0, cols - v.shape[0])).reshape(1, cols)


def _split_gnn_w(w, fin, fin_pad, fout_pad):
    ws = _pad2(w[:fin], fin_pad, fout_pad)
    wn = _pad2(w[fin:], fin_pad, fout_pad)
    return ws, wn


def kernel(x, enc_W0, enc_b0, enc_W1, enc_b1, enc_W2, enc_b2,
           venc_Wmu, venc_bmu, venc_Wlv, venc_blv,
           dec_W0, dec_b0, dec_W1, dec_b1, dec_W2, dec_b2,
           vdec_Wmu, vdec_bmu, vdec_Wsig, vdec_bsig):
    bsz = x.shape[0]

    es = [512, 426, 341, 256]
    ep = [512, 512, 384, 256]
    ds = [256, 341, 426, 512]
    dp = [256, 384, 512, 512]

    e0s_, e0n_ = _split_gnn_w(enc_W0, es[0], ep[0], ep[1])
    e1s_, e1n_ = _split_gnn_w(enc_W1, es[1], ep[1], ep[2])
    e2s_, e2n_ = _split_gnn_w(enc_W2, es[2], ep[2], ep[3])
    d0s_, d0n_ = _split_gnn_w(dec_W0, ds[0], dp[0], dp[1])
    d1s_, d1n_ = _split_gnn_w(dec_W1, ds[1], dp[1], dp[2])
    d2s_, d2n_ = _split_gnn_w(dec_W2, ds[2], dp[2], dp[3])

    weights = [
        e0s_, e0n_, _pad_b(enc_b0, ep[1]),
        e1s_, e1n_, _pad_b(enc_b1, ep[2]),
        e2s_, e2n_, _pad_b(enc_b2, ep[3]),
        venc_Wmu, venc_bmu.reshape(1, _Z), venc_Wlv, venc_blv.reshape(1, _Z),
        d0s_, d0n_, _pad_b(dec_b0, dp[1]),
        d1s_, d1n_, _pad_b(dec_b1, dp[2]),
        d2s_, d2n_, _pad_b(dec_b2, dp[3]),
        vdec_Wmu, vdec_bmu.reshape(1, _F), vdec_Wsig, vdec_bsig.reshape(1, _F),
    ]

    eps = jax.random.normal(jax.random.key(42), (bsz, _N, _Z), dtype=jnp.float32)
    xT = jnp.transpose(x, (1, 0, 2))
    epsT = jnp.transpose(eps, (1, 0, 2))

    grid = (bsz // _BB,)
    w_specs = [pl.BlockSpec(w.shape, lambda i: (0, 0)) for w in weights]
    outT = pl.pallas_call(
        _net_kernel,
        grid=grid,
        in_specs=[
            pl.BlockSpec((_N, _BB, _F), lambda i: (0, i, 0)),
            pl.BlockSpec((_N, _BB, _Z), lambda i: (0, i, 0)),
            *w_specs,
        ],
        out_specs=pl.BlockSpec((_N, _BB, _F), lambda i: (0, i, 0)),
        out_shape=jax.ShapeDtypeStruct((_N, bsz, _F), jnp.float32),
        compiler_params=pltpu.CompilerParams(
            dimension_semantics=("parallel",),
        ),
    )(xT, epsT, *weights)
    return jnp.transpose(outT, (1, 0, 2))


# speedup vs baseline: 2.8796x; 1.0036x over previous
import jax
import jax.numpy as jnp
from jax.experimental import pallas as pl
from jax.experimental.pallas import tpu as pltpu

_N = 64
_B = 512
_F = 512
_Z = 256
_BB = 8


def _roll_sum(h):
    left = jnp.concatenate([h[1:], h[:1]], axis=0)
    right = jnp.concatenate([h[-1:], h[:-1]], axis=0)
    return left + right


def _net_kernel(x_ref, eps_ref,
                e0s, e0n, e0b, e1s, e1n, e1b, e2s, e2n, e2b,
                wmu, bmu, wlv, blv,
                d0s, d0n, d0b, d1s, d1n, d1b, d2s, d2n, d2b,
                wdm, bdm, wds, bds,
                out_ref):
    n, b, f = x_ref.shape
    m = n * b

    def mm(a, w_ref):
        return jnp.dot(a.astype(jnp.bfloat16), w_ref[...],
                       preferred_element_type=jnp.float32)

    def gnn(h, ws, wn, bias):
        k = h.shape[-1]
        s = _roll_sum(h) * 0.5
        y = mm(h.reshape(m, k), ws) + mm(s.reshape(m, k), wn)
        y = jnp.maximum(y + bias[...], 0.0)
        return y.reshape(n, b, y.shape[-1])

    h = x_ref[...]
    h = gnn(h, e0s, e0n, e0b)
    h = gnn(h, e1s, e1n, e1b)
    h = gnn(h, e2s, e2n, e2b)

    h2 = h.reshape(m, h.shape[-1])
    mu = mm(h2, wmu) + bmu[...]
    lv = mm(h2, wlv) + blv[...]
    z = mu + jnp.exp(0.5 * lv) * eps_ref[...].reshape(m, _Z)
    h = z.reshape(n, b, _Z)

    h = gnn(h, d0s, d0n, d0b)
    h = gnn(h, d1s, d1n, d1b)
    h = gnn(h, d2s, d2n, d2b)

    h2 = h.reshape(m, h.shape[-1])
    dmu = mm(h2, wdm) + bdm[...]
    dsg = jax.nn.softplus(mm(h2, wds) + bds[...])
    out_ref[...] = jnp.exp(dmu + 0.5 * dsg * dsg).reshape(n, b, f)


def _pad2(w, rows, cols):
    r, c = w.shape
    return jnp.pad(w, ((0, rows - r), (0, cols - c)))


def _pad_b(v, cols):
    return jnp.pad(v, (0, cols - v.shape[0])).reshape(1, cols)


def _split_gnn_w(w, fin, fin_pad, fout_pad):
    ws = _pad2(w[:fin], fin_pad, fout_pad)
    wn = _pad2(w[fin:], fin_pad, fout_pad)
    return ws, wn


def kernel(x, enc_W0, enc_b0, enc_W1, enc_b1, enc_W2, enc_b2,
           venc_Wmu, venc_bmu, venc_Wlv, venc_blv,
           dec_W0, dec_b0, dec_W1, dec_b1, dec_W2, dec_b2,
           vdec_Wmu, vdec_bmu, vdec_Wsig, vdec_bsig):
    bsz = x.shape[0]

    es = [512, 426, 341, 256]
    ep = [512, 512, 384, 256]
    ds = [256, 341, 426, 512]
    dp = [256, 384, 512, 512]

    e0s_, e0n_ = _split_gnn_w(enc_W0, es[0], ep[0], ep[1])
    e1s_, e1n_ = _split_gnn_w(enc_W1, es[1], ep[1], ep[2])
    e2s_, e2n_ = _split_gnn_w(enc_W2, es[2], ep[2], ep[3])
    d0s_, d0n_ = _split_gnn_w(dec_W0, ds[0], dp[0], dp[1])
    d1s_, d1n_ = _split_gnn_w(dec_W1, ds[1], dp[1], dp[2])
    d2s_, d2n_ = _split_gnn_w(dec_W2, ds[2], dp[2], dp[3])

    weights = [
        e0s_, e0n_, _pad_b(enc_b0, ep[1]),
        e1s_, e1n_, _pad_b(enc_b1, ep[2]),
        e2s_, e2n_, _pad_b(enc_b2, ep[3]),
        venc_Wmu, venc_bmu.reshape(1, _Z), venc_Wlv, venc_blv.reshape(1, _Z),
        d0s_, d0n_, _pad_b(dec_b0, dp[1]),
        d1s_, d1n_, _pad_b(dec_b1, dp[2]),
        d2s_, d2n_, _pad_b(dec_b2, dp[3]),
        vdec_Wmu, vdec_bmu.reshape(1, _F), vdec_Wsig, vdec_bsig.reshape(1, _F),
    ]

    weights = [w.astype(jnp.bfloat16) if w.ndim == 2 and w.shape[0] > 1 else w
               for w in weights]
    eps = jax.random.normal(jax.random.key(42), (bsz, _N, _Z), dtype=jnp.float32)
    xT = jnp.transpose(x, (1, 0, 2))
    epsT = jnp.transpose(eps, (1, 0, 2))

    grid = (bsz // _BB,)
    w_specs = [pl.BlockSpec(w.shape, lambda i: (0, 0)) for w in weights]
    outT = pl.pallas_call(
        _net_kernel,
        grid=grid,
        in_specs=[
            pl.BlockSpec((_N, _BB, _F), lambda i: (0, i, 0)),
            pl.BlockSpec((_N, _BB, _Z), lambda i: (0, i, 0)),
            *w_specs,
        ],
        out_specs=pl.BlockSpec((_N, _BB, _F), lambda i: (0, i, 0)),
        out_shape=jax.ShapeDtypeStruct((_N, bsz, _F), jnp.float32),
        compiler_params=pltpu.CompilerParams(
            dimension_semantics=("parallel",),
        ),
    )(xT, epsT, *weights)
    return jnp.transpose(outT, (1, 0, 2))


# speedup vs baseline: 3.3799x; 1.1737x over previous
import jax
import jax.numpy as jnp
from jax.experimental import pallas as pl
from jax.experimental.pallas import tpu as pltpu

_N = 64
_F = 512
_Z = 256
_BB = 8


def _roll_sum(h):
    left = jnp.concatenate([h[:, 1:], h[:, :1]], axis=1)
    right = jnp.concatenate([h[:, -1:], h[:, :-1]], axis=1)
    return left + right


def _net_kernel(x_ref, eps_ref,
                e0w, e0b, e1w, e1b, e2w, e2b,
                vew, veb,
                d0w, d0b, d1w, d1b, d2w, d2b,
                vdw, vdb,
                out_ref):
    b, n, f = x_ref.shape
    m = b * n

    def mm(a, w_ref):
        return jnp.dot(a.astype(jnp.bfloat16), w_ref[...],
                       preferred_element_type=jnp.float32)

    def gnn(h, w, bias, fout):
        k = h.shape[-1]
        g = mm(h.reshape(m, k), w)
        y = g[:, :fout] + 0.5 * _roll_sum(
            g[:, fout:].reshape(b, n, fout)).reshape(m, fout)
        return jnp.maximum(y + bias[...], 0.0).reshape(b, n, fout)

    h = x_ref[...]
    h = gnn(h, e0w, e0b, 512)
    h = gnn(h, e1w, e1b, 384)
    h = gnn(h, e2w, e2b, 256)

    g = mm(h.reshape(m, _Z), vew) + veb[...]
    mu, lv = g[:, :_Z], g[:, _Z:]
    z = mu + jnp.exp(0.5 * lv) * eps_ref[...].reshape(m, _Z)
    h = z.reshape(b, n, _Z)

    h = gnn(h, d0w, d0b, 384)
    h = gnn(h, d1w, d1b, 512)
    h = gnn(h, d2w, d2b, 512)

    g = mm(h.reshape(m, _F), vdw) + vdb[...]
    dmu, pre = g[:, :_F], g[:, _F:]
    dsg = jax.nn.softplus(pre)
    out_ref[...] = jnp.exp(dmu + 0.5 * dsg * dsg).reshape(b, n, f)


def _pad2(w, rows, cols):
    r, c = w.shape
    return jnp.pad(w, ((0, rows - r), (0, cols - c)))


def _gnn_w(w, fin, fin_pad, fout_pad):
    ws = _pad2(w[:fin], fin_pad, fout_pad)
    wn = _pad2(w[fin:], fin_pad, fout_pad)
    return jnp.concatenate([ws, wn], axis=1)


def _bias(v, cols):
    return jnp.pad(v, (0, cols - v.shape[0])).reshape(1, cols)


def kernel(x, enc_W0, enc_b0, enc_W1, enc_b1, enc_W2, enc_b2,
           venc_Wmu, venc_bmu, venc_Wlv, venc_blv,
           dec_W0, dec_b0, dec_W1, dec_b1, dec_W2, dec_b2,
           vdec_Wmu, vdec_bmu, vdec_Wsig, vdec_bsig):
    bsz = x.shape[0]

    es = [512, 426, 341, 256]
    ep = [512, 512, 384, 256]
    ds = [256, 341, 426, 512]
    dp = [256, 384, 512, 512]

    weights = [
        _gnn_w(enc_W0, es[0], ep[0], ep[1]), _bias(enc_b0, ep[1]),
        _gnn_w(enc_W1, es[1], ep[1], ep[2]), _bias(enc_b1, ep[2]),
        _gnn_w(enc_W2, es[2], ep[2], ep[3]), _bias(enc_b2, ep[3]),
        jnp.concatenate([venc_Wmu, venc_Wlv], axis=1),
        jnp.concatenate([venc_bmu, venc_blv]).reshape(1, 2 * _Z),
        _gnn_w(dec_W0, ds[0], dp[0], dp[1]), _bias(dec_b0, dp[1]),
        _gnn_w(dec_W1, ds[1], dp[1], dp[2]), _bias(dec_b1, dp[2]),
        _gnn_w(dec_W2, ds[2], dp[2], dp[3]), _bias(dec_b2, dp[3]),
        jnp.concatenate([vdec_Wmu, vdec_Wsig], axis=1),
        jnp.concatenate([vdec_bmu, vdec_bsig]).reshape(1, 2 * _F),
    ]
    weights = [w.astype(jnp.bfloat16) if (w.ndim == 2 and w.shape[0] > 1) else w
               for w in weights]

    eps = jax.random.normal(jax.random.key(42), (bsz, _N, _Z), dtype=jnp.float32)

    grid = (bsz // _BB,)
    w_specs = [pl.BlockSpec(w.shape, lambda i: (0, 0)) for w in weights]
    out = pl.pallas_call(
        _net_kernel,
        grid=grid,
        in_specs=[
            pl.BlockSpec((_BB, _N, _F), lambda i: (i, 0, 0)),
            pl.BlockSpec((_BB, _N, _Z), lambda i: (i, 0, 0)),
            *w_specs,
        ],
        out_specs=pl.BlockSpec((_BB, _N, _F), lambda i: (i, 0, 0)),
        out_shape=jax.ShapeDtypeStruct((bsz, _N, _F), jnp.float32),
        compiler_params=pltpu.CompilerParams(
            dimension_semantics=("parallel",),
        ),
    )(x, eps, *weights)
    return out
